# baseline (speedup 1.0000x reference)
import functools

import jax
import jax.numpy as jnp
import numpy as np
from jax import lax
from jax.experimental import pallas as pl
from jax.experimental.pallas import tpu as pltpu
from jax.experimental.pallas import tpu_sc as plsc

_MAX_DEGREE = 10000.0
_K = 128
_L = 16
_NC = 2
_NS = 16
_NW = _NC * _NS

_QV = np.linspace(0.0, _MAX_DEGREE, _K).astype(np.float32)
_CS = np.cumsum(_QV, dtype=np.float32)
_RS = np.concatenate([_QV[1:], _QV[:1]]).astype(np.float32)
_RINV = (1.0 / (_RS + np.float32(1e-10)).astype(np.float32)).astype(np.float32)


@functools.cache
def _build(b, s):
    n = b * s
    npw = n // _NW
    rows = b // _NW
    vpr = s // _L
    nv = npw // _L

    mesh = plsc.VectorSubcoreMesh(core_axis_name="c", subcore_axis_name="s")

    @functools.partial(
        pl.kernel,
        mesh=mesh,
        out_type=jax.ShapeDtypeStruct((b, 2, s), jnp.float32),
        scratch_types=[
            pltpu.VMEM((npw,), jnp.float32),
            pltpu.VMEM((2 * npw,), jnp.float32),
            pltpu.VMEM((_K,), jnp.float32),
            pltpu.VMEM((_K,), jnp.float32),
            pltpu.SemaphoreType.DMA,
            pltpu.SemaphoreType.DMA,
            pltpu.SemaphoreType.DMA,
            pltpu.SemaphoreType.DMA,
        ],
        compiler_params=pltpu.CompilerParams(needs_layout_passes=False),
    )
    def run(deg_hbm, cs_hbm, ri_hbm, out_hbm, d_buf, o_buf, cs_buf, ri_buf,
            sem_a, sem_b, sem_c, sem_o):
        wid = lax.axis_index("s") * _NC + lax.axis_index("c")
        base = wid * npw
        c1 = pltpu.async_copy(cs_hbm, cs_buf, sem_a)
        c2 = pltpu.async_copy(ri_hbm, ri_buf, sem_b)
        c3 = pltpu.async_copy(deg_hbm.at[pl.ds(base, npw)], d_buf, sem_c)
        c1.wait()
        c2.wait()
        c3.wait()

        @plsc.parallel_loop(0, nv, step=1, unroll=8)
        def body(j):
            d = d_buf[pl.ds(j * _L, _L)]
            ge = d >= float(_CS[64])
            k = jnp.where(ge, 64, 0)
            sv = jnp.where(ge, float(_CS[64]), 0.0)
            v = jnp.where(ge, float(_CS[96]), float(_CS[32]))
            ge = d >= v
            k = jnp.where(ge, k + 32, k)
            sv = jnp.where(ge, v, sv)
            for st in (16, 8, 4, 2, 1):
                cand = k + st
                v = plsc.load_gather(cs_buf, [cand])
                ge = d >= v
                k = jnp.where(ge, cand, k)
                sv = jnp.where(ge, v, sv)
            start = sv
            rinv = plsc.load_gather(ri_buf, [k])
            idx = jnp.where(d >= 0.0, k, -1)
            rate = (d - start) * rinv
            rate = jnp.minimum(jnp.maximum(rate, 0.0), 1.0)
            off = (j // vpr) * (2 * s) + (j % vpr) * _L
            o_buf[pl.ds(off, _L)] = idx.astype(jnp.float32)
            o_buf[pl.ds(off + s, _L)] = rate

        out_copies = []
        for r in range(rows):
            row = wid * rows + r
            out_copies.append(pltpu.async_copy(
                o_buf.at[pl.ds(r * 2 * s, s)], out_hbm.at[row, 0], sem_o))
            out_copies.append(pltpu.async_copy(
                o_buf.at[pl.ds(r * 2 * s + s, s)], out_hbm.at[row, 1], sem_o))
        for c in out_copies:
            c.wait()

    return run


def kernel(degrees):
    b, s, _ = degrees.shape
    flat = degrees.reshape(b * s)
    out = _build(b, s)(flat, jnp.asarray(_CS), jnp.asarray(_RINV))
    return jnp.swapaxes(out, 1, 2)


# speedup vs baseline: 1.1090x; 1.1090x over previous
import functools

import jax
import jax.numpy as jnp
import numpy as np
from jax import lax
from jax.experimental import pallas as pl
from jax.experimental.pallas import tpu as pltpu
from jax.experimental.pallas import tpu_sc as plsc

_MAX_DEGREE = 10000.0
_K = 128
_L = 16
_NC = 2
_NS = 16
_NW = _NC * _NS

_QV = np.linspace(0.0, _MAX_DEGREE, _K).astype(np.float32)
_CS = np.cumsum(_QV, dtype=np.float32)
_RS = np.concatenate([_QV[1:], _QV[:1]]).astype(np.float32)
_RINV = (1.0 / (_RS + np.float32(1e-10)).astype(np.float32)).astype(np.float32)


@functools.cache
def _build(b, s):
    n = b * s
    npw = n // _NW
    rows = b // _NW
    vpr = s // _L
    nv = npw // _L

    mesh = plsc.VectorSubcoreMesh(core_axis_name="c", subcore_axis_name="s")

    @functools.partial(
        pl.kernel,
        mesh=mesh,
        out_type=jax.ShapeDtypeStruct((b, 2, s), jnp.float32),
        scratch_types=[
            pltpu.VMEM((npw,), jnp.float32),
            pltpu.VMEM((2 * npw,), jnp.float32),
            pltpu.VMEM((_K,), jnp.float32),
            pltpu.VMEM((_K,), jnp.float32),
            pltpu.SemaphoreType.DMA,
            pltpu.SemaphoreType.DMA,
            pltpu.SemaphoreType.DMA,
            pltpu.SemaphoreType.DMA,
        ],
        compiler_params=pltpu.CompilerParams(needs_layout_passes=False),
    )
    def run(deg_hbm, cs_hbm, ri_hbm, out_hbm, d_buf, o_buf, cs_buf, ri_buf,
            sem_a, sem_b, sem_c, sem_o):
        wid = lax.axis_index("s") * _NC + lax.axis_index("c")
        base = wid * npw
        c1 = pltpu.async_copy(cs_hbm, cs_buf, sem_a)
        c2 = pltpu.async_copy(ri_hbm, ri_buf, sem_b)
        c3 = pltpu.async_copy(deg_hbm.at[pl.ds(base, npw)], d_buf, sem_c)
        c1.wait()
        c2.wait()
        c3.wait()

        @plsc.parallel_loop(0, nv, step=1, unroll=8)
        def body(j):
            d = d_buf[pl.ds(j * _L, _L)]
            ge = d >= float(_CS[64])
            k = jnp.where(ge, 64, 0)
            v = jnp.where(ge, float(_CS[96]), float(_CS[32]))
            k = jnp.where(d >= v, k + 32, k)
            for st in (16, 8, 4, 2, 1):
                cand = k + st
                v = plsc.load_gather(cs_buf, [cand])
                k = jnp.where(d >= v, cand, k)
            start = plsc.load_gather(cs_buf, [k])
            rinv = plsc.load_gather(ri_buf, [k])
            idx = jnp.where(d >= 0.0, k, -1)
            rate = (d - start) * rinv
            rate = jnp.minimum(jnp.maximum(rate, 0.0), 1.0)
            off = (j // vpr) * (2 * s) + (j % vpr) * _L
            o_buf[pl.ds(off, _L)] = idx.astype(jnp.float32)
            o_buf[pl.ds(off + s, _L)] = rate

        out_copies = []
        for r in range(rows):
            row = wid * rows + r
            out_copies.append(pltpu.async_copy(
                o_buf.at[pl.ds(r * 2 * s, s)], out_hbm.at[row, 0], sem_o))
            out_copies.append(pltpu.async_copy(
                o_buf.at[pl.ds(r * 2 * s + s, s)], out_hbm.at[row, 1], sem_o))
        for c in out_copies:
            c.wait()

    return run


def kernel(degrees):
    b, s, _ = degrees.shape
    flat = degrees.reshape(b * s)
    out = _build(b, s)(flat, jnp.asarray(_CS), jnp.asarray(_RINV))
    return jnp.swapaxes(out, 1, 2)
